# Initial kernel scaffold; baseline (speedup 1.0000x reference)
import jax
import jax.numpy as jnp
from jax import lax
from jax.experimental import pallas as pl
from jax.experimental.pallas import tpu as pltpu
from jax.experimental.pallas import tpu_sc as plsc

BATCH = 4096
HIST = 200
EMB_DIM = 300
LANES = 16
NUM_CORES = 2
NUM_SUBCORES = 16
NUM_WORKERS = NUM_CORES * NUM_SUBCORES
B_PER_W = BATCH // NUM_WORKERS
CHUNK = 100

_OFFS = tuple(range(0, EMB_DIM - LANES, LANES)) + (EMB_DIM - LANES,)


def _body(x_hbm, table_hbm, out_hbm, idx_v, rows_v, out_v, sem):
    wid = lax.axis_index("s") * NUM_CORES + lax.axis_index("c")
    base = wid * B_PER_W

    pltpu.sync_copy(x_hbm.at[pl.ds(base, B_PER_W)], idx_v)

    inv_n = jnp.float32(1.0 / HIST)

    def row_body(r, carry):
        cp0 = pltpu.async_copy(
            table_hbm.at[idx_v.at[r, pl.ds(0, CHUNK)]],
            rows_v.at[pl.ds(0, CHUNK)], sem)
        cp1 = pltpu.async_copy(
            table_hbm.at[idx_v.at[r, pl.ds(CHUNK, CHUNK)]],
            rows_v.at[pl.ds(CHUNK, CHUNK)], sem)
        cp0.wait()
        cp1.wait()

        def acc_body(j, accs):
            return tuple(
                acc + rows_v[j, pl.ds(off, LANES)]
                for acc, off in zip(accs, _OFFS))

        accs0 = tuple(jnp.zeros((LANES,), jnp.float32) for _ in _OFFS)
        accs = lax.fori_loop(0, HIST, acc_body, accs0)
        for acc, off in zip(accs, _OFFS):
            out_v[r, pl.ds(off, LANES)] = acc * inv_n
        return carry

    lax.fori_loop(0, B_PER_W, row_body, 0)

    pltpu.sync_copy(out_v, out_hbm.at[pl.ds(base, B_PER_W)])


@jax.jit
def _mean_embedding(x, table):
    mesh = plsc.VectorSubcoreMesh(
        core_axis_name="c", subcore_axis_name="s")
    return pl.kernel(
        _body,
        out_type=jax.ShapeDtypeStruct((BATCH, EMB_DIM), jnp.float32),
        mesh=mesh,
        scratch_types=[
            pltpu.VMEM((B_PER_W, HIST), jnp.int32),
            pltpu.VMEM((HIST, EMB_DIM), jnp.float32),
            pltpu.VMEM((B_PER_W, EMB_DIM), jnp.float32),
            pltpu.SemaphoreType.DMA,
        ],
    )(x, table)


def kernel(x, table):
    return _mean_embedding(x, table)


# speedup vs baseline: 1.5752x; 1.5752x over previous
import jax
import jax.numpy as jnp
from jax import lax
from jax.experimental import pallas as pl
from jax.experimental.pallas import tpu as pltpu
from jax.experimental.pallas import tpu_sc as plsc

BATCH = 4096
HIST = 200
EMB_DIM = 300
EMB_PAD = 304
LANES = 16
NUM_CORES = 2
NUM_SUBCORES = 16
NUM_WORKERS = NUM_CORES * NUM_SUBCORES
B_PER_W = BATCH // NUM_WORKERS
CHUNK_A = 104
CHUNK_B = HIST - CHUNK_A

_OFFS = tuple(range(0, EMB_DIM - LANES, LANES)) + (EMB_DIM - LANES,)


def _body(x_hbm, table_hbm, out_hbm, idx_v, rows_v, out_v, sem):
    wid = lax.axis_index("s") * NUM_CORES + lax.axis_index("c")
    base = wid * B_PER_W

    pltpu.sync_copy(x_hbm.at[pl.ds(base, B_PER_W)], idx_v)

    inv_n = jnp.float32(1.0 / HIST)

    def row_body(r, carry):
        cp0 = pltpu.async_copy(
            table_hbm.at[idx_v.at[r, pl.ds(0, CHUNK_A)]],
            rows_v.at[pl.ds(0, CHUNK_A)], sem)
        cp1 = pltpu.async_copy(
            table_hbm.at[idx_v.at[r, pl.ds(CHUNK_A, CHUNK_B)]],
            rows_v.at[pl.ds(CHUNK_A, CHUNK_B)], sem)
        cp0.wait()
        cp1.wait()

        def acc_body(j, accs):
            return tuple(
                acc + rows_v[j, pl.ds(off, LANES)]
                for acc, off in zip(accs, _OFFS))

        accs0 = tuple(jnp.zeros((LANES,), jnp.float32) for _ in _OFFS)
        accs = lax.fori_loop(0, HIST, acc_body, accs0)
        for acc, off in zip(accs, _OFFS):
            out_v[r, pl.ds(off, LANES)] = acc * inv_n
        return carry

    lax.fori_loop(0, B_PER_W, row_body, 0)

    pltpu.sync_copy(out_v, out_hbm.at[pl.ds(base, B_PER_W)])


@jax.jit
def _mean_embedding(x, table):
    table_p = jnp.pad(table, ((0, 0), (0, EMB_PAD - EMB_DIM)))
    mesh = plsc.VectorSubcoreMesh(
        core_axis_name="c", subcore_axis_name="s")
    return pl.kernel(
        _body,
        out_type=jax.ShapeDtypeStruct((BATCH, EMB_DIM), jnp.float32),
        mesh=mesh,
        scratch_types=[
            pltpu.VMEM((B_PER_W, HIST), jnp.int32),
            pltpu.VMEM((HIST, EMB_PAD), jnp.float32),
            pltpu.VMEM((B_PER_W, EMB_DIM), jnp.float32),
            pltpu.SemaphoreType.DMA,
        ],
        compiler_params=pltpu.CompilerParams(use_tc_tiling_on_sc=False),
    )(x, table_p)


def kernel(x, table):
    return _mean_embedding(x, table)


# speedup vs baseline: 1.9033x; 1.2083x over previous
import jax
import jax.numpy as jnp
from jax import lax
from jax.experimental import pallas as pl
from jax.experimental.pallas import tpu as pltpu
from jax.experimental.pallas import tpu_sc as plsc

BATCH = 4096
HIST = 200
EMB_DIM = 300
EMB_PAD = 304
LANES = 16
NUM_CORES = 2
NUM_SUBCORES = 16
NUM_WORKERS = NUM_CORES * NUM_SUBCORES
B_PER_W = BATCH // NUM_WORKERS
CHUNK_A = 104
CHUNK_B = HIST - CHUNK_A

_OFFS = tuple(range(0, EMB_DIM - LANES, LANES)) + (EMB_DIM - LANES,)


def _body(x_hbm, table_hbm, out_hbm, idx_v, buf_a, buf_b, out_v,
          sem_a, sem_b):
    wid = lax.axis_index("s") * NUM_CORES + lax.axis_index("c")
    base = wid * B_PER_W

    pltpu.sync_copy(x_hbm.at[pl.ds(base, B_PER_W)], idx_v)

    inv_n = jnp.float32(1.0 / HIST)

    def gather_a(r):
        return pltpu.make_async_copy(
            table_hbm.at[idx_v.at[r, pl.ds(0, CHUNK_A)]], buf_a, sem_a)

    def gather_b(r):
        return pltpu.make_async_copy(
            table_hbm.at[idx_v.at[r, pl.ds(CHUNK_A, CHUNK_B)]], buf_b, sem_b)

    def reduce_into(buf, n, accs):
        def acc_body(j, accs):
            return tuple(
                acc + buf[j, pl.ds(off, LANES)]
                for acc, off in zip(accs, _OFFS))
        return lax.fori_loop(0, n, acc_body, accs)

    gather_a(0).start()
    gather_b(0).start()

    def row_body(r, carry):
        zeros = tuple(jnp.zeros((LANES,), jnp.float32) for _ in _OFFS)

        gather_a(r).wait()
        accs = reduce_into(buf_a, CHUNK_A, zeros)

        @pl.when(r < B_PER_W - 1)
        def _():
            gather_a(r + 1).start()

        gather_b(r).wait()
        accs = reduce_into(buf_b, CHUNK_B, accs)

        @pl.when(r < B_PER_W - 1)
        def _():
            gather_b(r + 1).start()

        for acc, off in zip(accs, _OFFS):
            out_v[r, pl.ds(off, LANES)] = acc * inv_n
        return carry

    lax.fori_loop(0, B_PER_W, row_body, 0)

    pltpu.sync_copy(out_v, out_hbm.at[pl.ds(base, B_PER_W)])


@jax.jit
def _mean_embedding(x, table):
    table_p = jnp.pad(table, ((0, 0), (0, EMB_PAD - EMB_DIM)))
    mesh = plsc.VectorSubcoreMesh(
        core_axis_name="c", subcore_axis_name="s")
    return pl.kernel(
        _body,
        out_type=jax.ShapeDtypeStruct((BATCH, EMB_DIM), jnp.float32),
        mesh=mesh,
        scratch_types=[
            pltpu.VMEM((B_PER_W, HIST), jnp.int32),
            pltpu.VMEM((CHUNK_A, EMB_PAD), jnp.float32),
            pltpu.VMEM((CHUNK_B, EMB_PAD), jnp.float32),
            pltpu.VMEM((B_PER_W, EMB_DIM), jnp.float32),
            pltpu.SemaphoreType.DMA,
            pltpu.SemaphoreType.DMA,
        ],
        compiler_params=pltpu.CompilerParams(use_tc_tiling_on_sc=False),
    )(x, table_p)


def kernel(x, table):
    return _mean_embedding(x, table)


# speedup vs baseline: 1.9742x; 1.0373x over previous
import jax
import jax.numpy as jnp
from jax import lax
from jax.experimental import pallas as pl
from jax.experimental.pallas import tpu as pltpu
from jax.experimental.pallas import tpu_sc as plsc

BATCH = 4096
HIST = 200
EMB_DIM = 300
EMB_PAD = 384
LANES = 16
NUM_CORES = 2
NUM_SUBCORES = 16
NUM_WORKERS = NUM_CORES * NUM_SUBCORES
B_PER_W = BATCH // NUM_WORKERS
CHUNK_A = 104
CHUNK_B = HIST - CHUNK_A
OUT_BLK = 32
N_BLKS = B_PER_W // OUT_BLK

_OFFS = tuple(range(0, 304, LANES))


def _body(x_hbm, table_hbm, out_hbm, idx_v, buf_a, buf_b, out_v,
          sem_a, sem_b):
    wid = lax.axis_index("s") * NUM_CORES + lax.axis_index("c")
    base = wid * B_PER_W * HIST

    pltpu.sync_copy(x_hbm.at[pl.ds(base, B_PER_W * HIST)], idx_v)

    inv_n = jnp.float32(1.0 / HIST)

    def gather_a(r):
        return pltpu.make_async_copy(
            table_hbm.at[idx_v.at[pl.ds(r * HIST, CHUNK_A)]], buf_a, sem_a)

    def gather_b(r):
        return pltpu.make_async_copy(
            table_hbm.at[idx_v.at[pl.ds(r * HIST + CHUNK_A, CHUNK_B)]],
            buf_b, sem_b)

    def reduce_into(buf, n, accs):
        def acc_body(j, accs):
            return tuple(
                acc + buf[j, pl.ds(off, LANES)]
                for acc, off in zip(accs, _OFFS))
        return lax.fori_loop(0, n, acc_body, accs)

    gather_a(0).start()
    gather_b(0).start()

    for blk in range(N_BLKS):
        last_blk = blk == N_BLKS - 1

        def row_body(r, carry, blk=blk, last_blk=last_blk):
            row = blk * OUT_BLK + r
            zeros = tuple(jnp.zeros((LANES,), jnp.float32) for _ in _OFFS)

            gather_a(row).wait()
            accs = reduce_into(buf_a, CHUNK_A, zeros)

            if last_blk:
                @pl.when(r < OUT_BLK - 1)
                def _():
                    gather_a(row + 1).start()
            else:
                gather_a(row + 1).start()

            gather_b(row).wait()
            accs = reduce_into(buf_b, CHUNK_B, accs)

            if last_blk:
                @pl.when(r < OUT_BLK - 1)
                def _():
                    gather_b(row + 1).start()
            else:
                gather_b(row + 1).start()

            for acc, off in zip(accs, _OFFS):
                out_v[r, pl.ds(off, LANES)] = acc * inv_n
            return carry

        lax.fori_loop(0, OUT_BLK, row_body, 0)
        pltpu.sync_copy(
            out_v, out_hbm.at[pl.ds(wid * B_PER_W + blk * OUT_BLK, OUT_BLK)])


@jax.jit
def _mean_embedding(x, table):
    table_p = jnp.pad(table, ((0, 0), (0, EMB_PAD - EMB_DIM)))
    xf = x.reshape(-1)
    mesh = plsc.VectorSubcoreMesh(
        core_axis_name="c", subcore_axis_name="s")
    out = pl.kernel(
        _body,
        out_type=jax.ShapeDtypeStruct((BATCH, EMB_PAD), jnp.float32),
        mesh=mesh,
        scratch_types=[
            pltpu.VMEM((B_PER_W * HIST,), jnp.int32),
            pltpu.VMEM((CHUNK_A, EMB_PAD), jnp.float32),
            pltpu.VMEM((CHUNK_B, EMB_PAD), jnp.float32),
            pltpu.VMEM((OUT_BLK, EMB_PAD), jnp.float32),
            pltpu.SemaphoreType.DMA,
            pltpu.SemaphoreType.DMA,
        ],
    )(xf, table_p)
    return out[:, :EMB_DIM]


def kernel(x, table):
    return _mean_embedding(x, table)


# speedup vs baseline: 2.8283x; 1.4326x over previous
import jax
import jax.numpy as jnp
from jax import lax
from jax.experimental import pallas as pl
from jax.experimental.pallas import tpu as pltpu
from jax.experimental.pallas import tpu_sc as plsc

BATCH = 4096
HIST = 200
EMB_DIM = 300
EMB_PAD = 384
LANES = 16
NUM_CORES = 2
NUM_SUBCORES = 16
NUM_WORKERS = NUM_CORES * NUM_SUBCORES
B_PER_W = BATCH // NUM_WORKERS
CHUNK_A = 104
CHUNK_B = HIST - CHUNK_A
OUT_BLK = 32
N_BLKS = B_PER_W // OUT_BLK

_OFFS = tuple(range(0, EMB_DIM - LANES, LANES)) + (EMB_DIM - LANES,)


def _body(x_hbm, table_hbm, out_hbm, idx_v, buf_a, buf_b, out_v,
          sem_a, sem_b):
    wid = lax.axis_index("s") * NUM_CORES + lax.axis_index("c")
    base = wid * B_PER_W * HIST

    pltpu.sync_copy(x_hbm.at[pl.ds(base, B_PER_W * HIST)], idx_v)

    inv_n = jnp.float32(1.0 / HIST)

    def gather_a(r):
        return pltpu.make_async_copy(
            table_hbm.at[idx_v.at[pl.ds(r * HIST, CHUNK_A)]], buf_a, sem_a)

    def gather_b(r):
        return pltpu.make_async_copy(
            table_hbm.at[idx_v.at[pl.ds(r * HIST + CHUNK_A, CHUNK_B)]],
            buf_b, sem_b)

    def reduce_into(buf, n, accs):
        def acc_body(j, accs):
            return tuple(
                acc + buf[j, pl.ds(off, LANES)]
                for acc, off in zip(accs, _OFFS))
        return lax.fori_loop(0, n, acc_body, accs)

    gather_a(0).start()
    gather_b(0).start()

    for blk in range(N_BLKS):
        last_blk = blk == N_BLKS - 1

        def row_body(r, carry, blk=blk, last_blk=last_blk):
            row = blk * OUT_BLK + r
            zeros = tuple(jnp.zeros((LANES,), jnp.float32) for _ in _OFFS)

            gather_a(row).wait()
            accs = reduce_into(buf_a, CHUNK_A, zeros)

            if last_blk:
                @pl.when(r < OUT_BLK - 1)
                def _():
                    gather_a(row + 1).start()
            else:
                gather_a(row + 1).start()

            gather_b(row).wait()
            accs = reduce_into(buf_b, CHUNK_B, accs)

            if last_blk:
                @pl.when(r < OUT_BLK - 1)
                def _():
                    gather_b(row + 1).start()
            else:
                gather_b(row + 1).start()

            for acc, off in zip(accs, _OFFS):
                out_v[r, pl.ds(off, LANES)] = acc * inv_n
            return carry

        lax.fori_loop(0, OUT_BLK, row_body, 0)
        pltpu.sync_copy(
            out_v, out_hbm.at[pl.ds(wid * B_PER_W + blk * OUT_BLK, OUT_BLK)])


PAD_BLK = 2000


def _pad_body(t_ref, o_ref):
    o_ref[:, :EMB_DIM] = t_ref[...]
    o_ref[:, EMB_DIM:] = jnp.zeros((PAD_BLK, EMB_PAD - EMB_DIM), jnp.float32)


def _pad_table(table):
    v = table.shape[0]
    return pl.pallas_call(
        _pad_body,
        grid=(v // PAD_BLK,),
        in_specs=[pl.BlockSpec((PAD_BLK, EMB_DIM), lambda i: (i, 0))],
        out_specs=pl.BlockSpec((PAD_BLK, EMB_PAD), lambda i: (i, 0)),
        out_shape=jax.ShapeDtypeStruct((v, EMB_PAD), jnp.float32),
    )(table)


@jax.jit
def _mean_embedding(x, table):
    table_p = _pad_table(table)
    xf = x.reshape(-1)
    mesh = plsc.VectorSubcoreMesh(
        core_axis_name="c", subcore_axis_name="s")
    return pl.kernel(
        _body,
        out_type=jax.ShapeDtypeStruct((BATCH, EMB_DIM), jnp.float32),
        mesh=mesh,
        scratch_types=[
            pltpu.VMEM((B_PER_W * HIST,), jnp.int32),
            pltpu.VMEM((CHUNK_A, EMB_PAD), jnp.float32),
            pltpu.VMEM((CHUNK_B, EMB_PAD), jnp.float32),
            pltpu.VMEM((OUT_BLK, EMB_DIM), jnp.float32),
            pltpu.SemaphoreType.DMA,
            pltpu.SemaphoreType.DMA,
        ],
    )(xf, table_p)


def kernel(x, table):
    return _mean_embedding(x, table)
